# Initial kernel scaffold; baseline (speedup 1.0000x reference)
import jax
import jax.numpy as jnp
from jax.experimental import pallas as pl


def kernel(x, edge_index, W1, W2, W3):
    raise NotImplementedError("write your pallas kernel here")


# speedup vs baseline: 10.4002x; 10.4002x over previous
import functools

import jax
import jax.numpy as jnp
from jax import lax
from jax.experimental import pallas as pl
from jax.experimental.pallas import tpu as pltpu
from jax.experimental.pallas import tpu_sc as plsc

NC = 2
NS = 16
NW = NC * NS
CHUNK = 128


def _round_up(a, b):
  return (a + b - 1) // b * b


def _make_deg_kernel(Np, dcpt):
  mesh = plsc.VectorSubcoreMesh(core_axis_name="c", subcore_axis_name="s")
  tp = 2 * Np // NS

  @functools.partial(
      pl.kernel,
      out_type=jax.ShapeDtypeStruct((NC * 2 * Np,), jnp.float32),
      mesh=mesh,
      scratch_types=[
          pltpu.VMEM((dcpt, CHUNK), jnp.int32),
          pltpu.VMEM((CHUNK,), jnp.float32),
          pltpu.VMEM((tp,), jnp.float32),
          pltpu.VMEM_SHARED((2 * Np,), jnp.float32),
      ],
  )
  def deg_kernel(idx_hbm, out_hbm, idx_v, ones_v, stage_v, deg_sh):
    c = lax.axis_index("c")
    s = lax.axis_index("s")
    w = c * NS + s
    off = pl.multiple_of(s * tp, 8)
    pltpu.sync_copy(idx_hbm.at[w], idx_v)
    for k in range(CHUNK // 16):
      ones_v[pl.ds(k * 16, 16)] = jnp.full((16,), 1.0, jnp.float32)

    def zbody(j, carry):
      stage_v[pl.ds(j * 16, 16)] = jnp.zeros((16,), jnp.float32)
      return carry

    lax.fori_loop(0, tp // 16, zbody, 0)
    pltpu.sync_copy(stage_v, deg_sh.at[pl.ds(off, tp)])
    plsc.subcore_barrier()

    def body(j, carry):
      pltpu.sync_copy(ones_v, deg_sh.at[idx_v.at[j]], add=True)
      return carry

    lax.fori_loop(0, dcpt, body, 0)
    plsc.subcore_barrier()
    out_off = pl.multiple_of(c * (2 * Np) + s * tp, 8)
    pltpu.sync_copy(deg_sh.at[pl.ds(off, tp)], stage_v)
    pltpu.sync_copy(stage_v, out_hbm.at[pl.ds(out_off, tp)])

  return deg_kernel


def _make_prop_kernel(Np, H, cpt):
  mesh = plsc.VectorSubcoreMesh(core_axis_name="c", subcore_axis_name="s")
  rpt = Np // NS

  @functools.partial(
      pl.kernel,
      out_type=jax.ShapeDtypeStruct((NC * Np, H), jnp.float32),
      mesh=mesh,
      compiler_params=pltpu.CompilerParams(use_tc_tiling_on_sc=False),
      scratch_types=[
          pltpu.VMEM((cpt, CHUNK), jnp.int32),
          pltpu.VMEM((cpt, CHUNK), jnp.int32),
          pltpu.VMEM((2, CHUNK, H), jnp.float32),
          pltpu.VMEM((rpt, H), jnp.float32),
          pltpu.VMEM_SHARED((Np, H), jnp.float32),
          pltpu.SemaphoreType.DMA,
          pltpu.SemaphoreType.DMA,
      ],
  )
  def prop_kernel(table_hbm, src_hbm, dst_hbm, out_hbm,
                  src_v, dst_v, rows_v, stage_v, acc_sh, sem_g0, sem_g1):
    c = lax.axis_index("c")
    s = lax.axis_index("s")
    w = c * NS + s
    roff = pl.multiple_of(s * rpt, 8)
    pltpu.sync_copy(src_hbm.at[w], src_v)
    pltpu.sync_copy(dst_hbm.at[w], dst_v)

    def zbody(j, carry):
      for k in range(H // 16):
        stage_v[j, pl.ds(k * 16, 16)] = jnp.zeros((16,), jnp.float32)
      return carry

    lax.fori_loop(0, rpt, zbody, 0)
    pltpu.sync_copy(stage_v, acc_sh.at[pl.ds(roff, rpt)])
    plsc.subcore_barrier()

    sems = (sem_g0, sem_g1)
    pltpu.async_copy(table_hbm.at[src_v.at[0]], rows_v.at[0], sems[0])

    def body(jo, carry):
      for b in range(2):
        jj = 2 * jo + b

        @pl.when(jj < cpt)
        def _():
          @pl.when(jj + 1 < cpt)
          def _():
            pltpu.async_copy(table_hbm.at[src_v.at[jj + 1]],
                             rows_v.at[1 - b],
                             sems[1 - b])

          pltpu.make_async_copy(table_hbm.at[src_v.at[jj]], rows_v.at[b],
                                sems[b]).wait()
          pltpu.sync_copy(rows_v.at[b], acc_sh.at[dst_v.at[jj]], add=True)

      return carry

    _ = lax.fori_loop(0, (cpt + 1) // 2, body, 0)
    plsc.subcore_barrier()
    pltpu.sync_copy(acc_sh.at[pl.ds(roff, rpt)], stage_v)
    out_off = pl.multiple_of(c * Np + s * rpt, 8)
    pltpu.sync_copy(stage_v, out_hbm.at[pl.ds(out_off, rpt)])

  return prop_kernel


def _prep_body(x_ref, w1_ref, deg_ref, g1_ref, ns_ref, nd_ref):
  d = deg_ref[...]
  nsrc = lax.rsqrt(jnp.maximum(d[0, 0] + d[1, 0], 1.0))
  ndst = lax.rsqrt(jnp.maximum(d[0, 1] + d[1, 1], 1.0))
  h = jnp.dot(x_ref[...], w1_ref[...], preferred_element_type=jnp.float32)
  g1_ref[...] = h * nsrc
  ns_ref[...] = nsrc
  nd_ref[...] = ndst


def _mid_body(p_ref, ns_ref, nd_ref, o_ref):
  agg = p_ref[0] + p_ref[1]
  h1 = jnp.maximum(agg * nd_ref[...], 0.0)
  o_ref[...] = h1 * ns_ref[...]


def _dec_body(q_ref, nd_ref, w2_ref, w3_ref, mu_ref, lv_ref):
  sblk = (q_ref[0] + q_ref[1]) * nd_ref[...]
  mu_ref[...] = jnp.dot(sblk, w2_ref[...], preferred_element_type=jnp.float32)
  lv_ref[...] = jnp.dot(sblk, w3_ref[...], preferred_element_type=jnp.float32)


def _adj_body(a_ref, b_ref, o_ref):
  o_ref[...] = jnp.dot(a_ref[...], b_ref[...], preferred_element_type=jnp.float32)


def kernel(x, edge_index, W1, W2, W3):
  N, D = x.shape
  E = edge_index.shape[1]
  H1 = W1.shape[1]
  H2 = W2.shape[1]

  Np = _round_up(N + 1, 128)
  pad_n = Np - N

  src = edge_index[0]
  dst = edge_index[1]

  de = 2 * E
  dcpt = _round_up(-(-de // NW), CHUNK) // CHUNK
  de_pad = NW * dcpt * CHUNK
  dpad = N + (jnp.arange(de_pad - de, dtype=jnp.int32) % pad_n)
  idx_cat = jnp.concatenate([src, dst + Np, dpad]).reshape(NW, dcpt, CHUNK)
  deg_parts = _make_deg_kernel(Np, dcpt)(idx_cat)
  deg4 = deg_parts.reshape(NC, 2, Np, 1)

  blk = 1264
  nblk = Np // blk
  x_pad = jnp.pad(x, ((0, Np - N), (0, 0)))
  g1, ns, nd = pl.pallas_call(
      _prep_body,
      grid=(nblk,),
      in_specs=[
          pl.BlockSpec((blk, D), lambda i: (i, 0)),
          pl.BlockSpec((D, H1), lambda i: (0, 0)),
          pl.BlockSpec((NC, 2, blk, 1), lambda i: (0, 0, i, 0)),
      ],
      out_specs=[
          pl.BlockSpec((blk, H1), lambda i: (i, 0)),
          pl.BlockSpec((blk, 1), lambda i: (i, 0)),
          pl.BlockSpec((blk, 1), lambda i: (i, 0)),
      ],
      out_shape=[
          jax.ShapeDtypeStruct((Np, H1), jnp.float32),
          jax.ShapeDtypeStruct((Np, 1), jnp.float32),
          jax.ShapeDtypeStruct((Np, 1), jnp.float32),
      ],
  )(x_pad, W1, deg4)

  cpt = _round_up(-(-E // NW), CHUNK) // CHUNK
  e_pad = NW * cpt * CHUNK
  epad = N + (jnp.arange(e_pad - E, dtype=jnp.int32) % pad_n)
  src_r = jnp.concatenate([src, epad]).reshape(NW, cpt, CHUNK)
  dst_r = jnp.concatenate([dst, epad]).reshape(NW, cpt, CHUNK)
  prop = _make_prop_kernel(Np, H1, cpt)
  agg_parts = prop(g1, src_r, dst_r).reshape(NC, Np, H1)

  h1n = pl.pallas_call(
      _mid_body,
      grid=(nblk,),
      in_specs=[
          pl.BlockSpec((NC, blk, H1), lambda i: (0, i, 0)),
          pl.BlockSpec((blk, 1), lambda i: (i, 0)),
          pl.BlockSpec((blk, 1), lambda i: (i, 0)),
      ],
      out_specs=pl.BlockSpec((blk, H1), lambda i: (i, 0)),
      out_shape=jax.ShapeDtypeStruct((Np, H1), jnp.float32),
  )(agg_parts, ns, nd)

  q_parts = prop(h1n, src_r, dst_r).reshape(NC, Np, H1)

  mu_pad, lv_pad = pl.pallas_call(
      _dec_body,
      grid=(nblk,),
      in_specs=[
          pl.BlockSpec((NC, blk, H1), lambda i: (0, i, 0)),
          pl.BlockSpec((blk, 1), lambda i: (i, 0)),
          pl.BlockSpec((H1, H2), lambda i: (0, 0)),
          pl.BlockSpec((H1, H2), lambda i: (0, 0)),
      ],
      out_specs=[
          pl.BlockSpec((blk, H2), lambda i: (i, 0)),
          pl.BlockSpec((blk, H2), lambda i: (i, 0)),
      ],
      out_shape=[
          jax.ShapeDtypeStruct((Np, H2), jnp.float32),
          jax.ShapeDtypeStruct((Np, H2), jnp.float32),
      ],
  )(q_parts, nd, W2, W3)

  mu = mu_pad[:N]
  logvar = lv_pad[:N]

  muT = jnp.transpose(mu)
  bm = 400
  adj = pl.pallas_call(
      _adj_body,
      grid=(N // bm,),
      in_specs=[
          pl.BlockSpec((bm, H2), lambda i: (i, 0)),
          pl.BlockSpec((H2, N), lambda i: (0, 0)),
      ],
      out_specs=pl.BlockSpec((bm, N), lambda i: (i, 0)),
      out_shape=jax.ShapeDtypeStruct((N, N), jnp.float32),
  )(mu, muT)

  return (adj, mu, logvar)


# speedup vs baseline: 10.4678x; 1.0065x over previous
import functools

import jax
import jax.numpy as jnp
from jax import lax
from jax.experimental import pallas as pl
from jax.experimental.pallas import tpu as pltpu
from jax.experimental.pallas import tpu_sc as plsc

NC = 2
NS = 16
NW = NC * NS
CHUNK = 128


def _round_up(a, b):
  return (a + b - 1) // b * b


def _make_deg_kernel(Np, nch):
  mesh = plsc.VectorSubcoreMesh(core_axis_name="c", subcore_axis_name="s")
  maxcpt = -(-nch // NW)
  tp = Np // NS
  tpz = _round_up(tp, 16)

  @functools.partial(
      pl.kernel,
      out_type=jax.ShapeDtypeStruct((NC * 2 * Np,), jnp.float32),
      mesh=mesh,
      compiler_params=pltpu.CompilerParams(use_tc_tiling_on_sc=False),
      scratch_types=[
          pltpu.VMEM((maxcpt, CHUNK), jnp.int32),
          pltpu.VMEM((maxcpt, CHUNK), jnp.int32),
          pltpu.VMEM((CHUNK,), jnp.float32),
          pltpu.VMEM((tpz,), jnp.float32),
          pltpu.VMEM_SHARED((Np,), jnp.float32),
          pltpu.VMEM_SHARED((Np,), jnp.float32),
      ],
  )
  def deg_kernel(src_hbm, dst_hbm, out_hbm, src_v, dst_v, ones_v, stage_v,
                 dega_sh, degb_sh):
    c = lax.axis_index("c")
    s = lax.axis_index("s")
    w = c * NS + s
    cs = (nch * w) // NW
    ce = (nch * (w + 1)) // NW
    t = ce - cs
    pltpu.sync_copy(src_hbm.at[pl.ds(cs, maxcpt)], src_v)
    pltpu.sync_copy(dst_hbm.at[pl.ds(cs, maxcpt)], dst_v)
    for k in range(CHUNK // 16):
      ones_v[pl.ds(k * 16, 16)] = jnp.full((16,), 1.0, jnp.float32)

    def zbody(j, carry):
      stage_v[pl.ds(j * 16, 16)] = jnp.zeros((16,), jnp.float32)
      return carry

    lax.fori_loop(0, tpz // 16, zbody, 0)
    off = pl.multiple_of(s * tp, 8)
    pltpu.sync_copy(stage_v.at[pl.ds(0, tp)], dega_sh.at[pl.ds(off, tp)])
    pltpu.sync_copy(stage_v.at[pl.ds(0, tp)], degb_sh.at[pl.ds(off, tp)])
    plsc.subcore_barrier()

    def body(j, carry):
      @pl.when(j < t)
      def _():
        pltpu.sync_copy(ones_v, dega_sh.at[src_v.at[j]], add=True)
        pltpu.sync_copy(ones_v, degb_sh.at[dst_v.at[j]], add=True)

      return carry

    lax.fori_loop(0, maxcpt, body, 0)
    plsc.subcore_barrier()
    base = c * 2 * Np
    pltpu.sync_copy(dega_sh.at[pl.ds(off, tp)], stage_v.at[pl.ds(0, tp)])
    pltpu.sync_copy(stage_v.at[pl.ds(0, tp)],
                    out_hbm.at[pl.ds(pl.multiple_of(base + s * tp, 8), tp)])
    pltpu.sync_copy(degb_sh.at[pl.ds(off, tp)], stage_v.at[pl.ds(0, tp)])
    pltpu.sync_copy(stage_v.at[pl.ds(0, tp)],
                    out_hbm.at[pl.ds(pl.multiple_of(base + Np + s * tp, 8), tp)])

  return deg_kernel


def _make_prop_kernel(Np, H, nch):
  mesh = plsc.VectorSubcoreMesh(core_axis_name="c", subcore_axis_name="s")
  maxcpt = -(-nch // NW)
  rpt = Np // NS

  @functools.partial(
      pl.kernel,
      out_type=jax.ShapeDtypeStruct((NC * Np, H), jnp.float32),
      mesh=mesh,
      compiler_params=pltpu.CompilerParams(use_tc_tiling_on_sc=False),
      scratch_types=[
          pltpu.VMEM((maxcpt, CHUNK), jnp.int32),
          pltpu.VMEM((maxcpt, CHUNK), jnp.int32),
          pltpu.VMEM((4, CHUNK, H), jnp.float32),
          pltpu.VMEM_SHARED((Np, H), jnp.float32),
          pltpu.SemaphoreType.DMA,
          pltpu.SemaphoreType.DMA,
          pltpu.SemaphoreType.DMA,
          pltpu.SemaphoreType.DMA,
          pltpu.SemaphoreType.DMA,
          pltpu.SemaphoreType.DMA,
          pltpu.SemaphoreType.DMA,
          pltpu.SemaphoreType.DMA,
      ],
  )
  def prop_kernel(table_hbm, src_hbm, dst_hbm, out_hbm,
                  src_v, dst_v, rows_v, acc_sh,
                  sg0, sg1, sg2, sg3, ss0, ss1, ss2, ss3):
    c = lax.axis_index("c")
    s = lax.axis_index("s")
    w = c * NS + s
    cs = (nch * w) // NW
    ce = (nch * (w + 1)) // NW
    t = ce - cs
    sgs = (sg0, sg1, sg2, sg3)
    sss = (ss0, ss1, ss2, ss3)
    roff = pl.multiple_of(s * rpt, 8)
    nq = rpt // CHUNK
    tail = rpt - nq * CHUNK
    pltpu.sync_copy(src_hbm.at[pl.ds(cs, maxcpt)], src_v)
    pltpu.sync_copy(dst_hbm.at[pl.ds(cs, maxcpt)], dst_v)

    def zbody(j, carry):
      for k in range(H // 16):
        rows_v[0, j, pl.ds(k * 16, 16)] = jnp.zeros((16,), jnp.float32)
      return carry

    lax.fori_loop(0, CHUNK, zbody, 0)
    for q in range(nq):
      pltpu.sync_copy(rows_v.at[0],
                      acc_sh.at[pl.ds(pl.multiple_of(s * rpt + q * CHUNK, 8),
                                      CHUNK)])
    if tail:
      pltpu.sync_copy(rows_v.at[0, pl.ds(0, tail)],
                      acc_sh.at[pl.ds(pl.multiple_of(s * rpt + nq * CHUNK, 8),
                                      tail)])
    plsc.subcore_barrier()

    for b in range(2):
      @pl.when(b < t)
      def _(b=b):
        pltpu.async_copy(table_hbm.at[src_v.at[b]], rows_v.at[b], sgs[b])

    def body(jo, carry):
      for b in range(4):
        jj = 4 * jo + b
        pj = jj + 2
        pb = (b + 2) % 4

        @pl.when(pj < t)
        def _(pj=pj, pb=pb):
          @pl.when(pj >= 4)
          def _():
            pltpu.make_async_copy(rows_v.at[pb], acc_sh.at[dst_v.at[pj - 4]],
                                  sss[pb]).wait()

          pltpu.async_copy(table_hbm.at[src_v.at[pj]], rows_v.at[pb], sgs[pb])

        @pl.when(jj < t)
        def _(jj=jj, b=b):
          pltpu.make_async_copy(table_hbm.at[src_v.at[jj]], rows_v.at[b],
                                sgs[b]).wait()
          pltpu.async_copy(rows_v.at[b], acc_sh.at[dst_v.at[jj]], sss[b],
                           add=True)

      return carry

    _ = lax.fori_loop(0, (maxcpt + 3) // 4, body, 0)

    for b in range(4):
      @pl.when(b < t)
      def _(b=b):
        last = ((t - 1 - b) // 4) * 4 + b
        pltpu.make_async_copy(rows_v.at[b], acc_sh.at[dst_v.at[last]],
                              sss[b]).wait()

    plsc.subcore_barrier()

    obase = c * Np + s * rpt
    for q in range(nq):
      aq = pl.multiple_of(s * rpt + q * CHUNK, 8)
      pltpu.async_copy(acc_sh.at[pl.ds(aq, CHUNK)], rows_v.at[q], sgs[q])
    for q in range(nq):
      aq = pl.multiple_of(s * rpt + q * CHUNK, 8)
      pltpu.make_async_copy(acc_sh.at[pl.ds(aq, CHUNK)],
                            rows_v.at[q], sgs[q]).wait()
      oq = pl.multiple_of(obase + q * CHUNK, 8)
      pltpu.async_copy(rows_v.at[q], out_hbm.at[pl.ds(oq, CHUNK)], sss[q])
    if tail:
      o0 = pl.multiple_of(obase, 8)
      pltpu.make_async_copy(rows_v.at[0], out_hbm.at[pl.ds(o0, CHUNK)],
                            sss[0]).wait()
      pltpu.sync_copy(acc_sh.at[pl.ds(pl.multiple_of(s * rpt + nq * CHUNK, 8),
                                      tail)],
                      rows_v.at[0, pl.ds(0, tail)])
      ot = pl.multiple_of(obase + nq * CHUNK, 8)
      pltpu.sync_copy(rows_v.at[0, pl.ds(0, tail)],
                      out_hbm.at[pl.ds(ot, tail)])
    for q in range(1 if tail else 0, nq):
      oq = pl.multiple_of(obase + q * CHUNK, 8)
      pltpu.make_async_copy(rows_v.at[q], out_hbm.at[pl.ds(oq, CHUNK)],
                            sss[q]).wait()

  return prop_kernel


def _norms(deg_blk):
  nsrc = lax.rsqrt(jnp.maximum(deg_blk[0, 0] + deg_blk[1, 0], 1.0))
  ndst = lax.rsqrt(jnp.maximum(deg_blk[0, 1] + deg_blk[1, 1], 1.0))
  return nsrc, ndst


def _prep_body(x_ref, w1_ref, deg_ref, g1_ref):
  nsrc, _ = _norms(deg_ref[...])
  h = jnp.dot(x_ref[...], w1_ref[...], preferred_element_type=jnp.float32)
  g1_ref[...] = h * nsrc


def _mid_body(p_ref, deg_ref, o_ref):
  nsrc, ndst = _norms(deg_ref[...])
  agg = p_ref[0] + p_ref[1]
  h1 = jnp.maximum(agg * ndst, 0.0)
  o_ref[...] = h1 * nsrc


def _dec_body(q_ref, deg_ref, w2_ref, w3_ref, mu_ref, lv_ref):
  _, ndst = _norms(deg_ref[...])
  sblk = (q_ref[0] + q_ref[1]) * ndst
  mu_ref[...] = jnp.dot(sblk, w2_ref[...], preferred_element_type=jnp.float32)
  lv_ref[...] = jnp.dot(sblk, w3_ref[...], preferred_element_type=jnp.float32)


def _adj_body(a_ref, b_ref, o_ref):
  o_ref[...] = jnp.dot(a_ref[...], b_ref[...], preferred_element_type=jnp.float32)


def kernel(x, edge_index, W1, W2, W3):
  N, D = x.shape
  E = edge_index.shape[1]
  H1 = W1.shape[1]
  H2 = W2.shape[1]

  Np = _round_up(N, 128)
  nch = E // CHUNK
  assert E % CHUNK == 0

  src_r = edge_index[0].reshape(nch, CHUNK)
  dst_r = edge_index[1].reshape(nch, CHUNK)

  deg_parts = _make_deg_kernel(Np, nch)(src_r, dst_r)
  deg4 = deg_parts.reshape(NC, 2, Np, 1)

  blk = 400
  nblk = N // blk
  g1 = pl.pallas_call(
      _prep_body,
      grid=(nblk,),
      in_specs=[
          pl.BlockSpec((blk, D), lambda i: (i, 0)),
          pl.BlockSpec((D, H1), lambda i: (0, 0)),
          pl.BlockSpec((NC, 2, blk, 1), lambda i: (0, 0, i, 0)),
      ],
      out_specs=pl.BlockSpec((blk, H1), lambda i: (i, 0)),
      out_shape=jax.ShapeDtypeStruct((N, H1), jnp.float32),
  )(x, W1, deg4)

  prop = _make_prop_kernel(Np, H1, nch)
  agg_parts = prop(g1, src_r, dst_r).reshape(NC, Np, H1)

  h1n = pl.pallas_call(
      _mid_body,
      grid=(nblk,),
      in_specs=[
          pl.BlockSpec((NC, blk, H1), lambda i: (0, i, 0)),
          pl.BlockSpec((NC, 2, blk, 1), lambda i: (0, 0, i, 0)),
      ],
      out_specs=pl.BlockSpec((blk, H1), lambda i: (i, 0)),
      out_shape=jax.ShapeDtypeStruct((N, H1), jnp.float32),
  )(agg_parts, deg4)

  q_parts = prop(h1n, src_r, dst_r).reshape(NC, Np, H1)

  mu, logvar = pl.pallas_call(
      _dec_body,
      grid=(nblk,),
      in_specs=[
          pl.BlockSpec((NC, blk, H1), lambda i: (0, i, 0)),
          pl.BlockSpec((NC, 2, blk, 1), lambda i: (0, 0, i, 0)),
          pl.BlockSpec((H1, H2), lambda i: (0, 0)),
          pl.BlockSpec((H1, H2), lambda i: (0, 0)),
      ],
      out_specs=[
          pl.BlockSpec((blk, H2), lambda i: (i, 0)),
          pl.BlockSpec((blk, H2), lambda i: (i, 0)),
      ],
      out_shape=[
          jax.ShapeDtypeStruct((N, H2), jnp.float32),
          jax.ShapeDtypeStruct((N, H2), jnp.float32),
      ],
  )(q_parts, deg4, W2, W3)

  muT = jnp.transpose(mu)
  bm = 400
  adj = pl.pallas_call(
      _adj_body,
      grid=(N // bm,),
      in_specs=[
          pl.BlockSpec((bm, H2), lambda i: (i, 0)),
          pl.BlockSpec((H2, N), lambda i: (0, 0)),
      ],
      out_specs=pl.BlockSpec((bm, N), lambda i: (i, 0)),
      out_shape=jax.ShapeDtypeStruct((N, N), jnp.float32),
  )(mu, muT)

  return (adj, mu, logvar)


# speedup vs baseline: 12.6689x; 1.2103x over previous
import functools

import jax
import jax.numpy as jnp
from jax import lax
from jax.experimental import pallas as pl
from jax.experimental.pallas import tpu as pltpu
from jax.experimental.pallas import tpu_sc as plsc

NC = 2
NS = 16
NW = NC * NS
CHUNK = 128


def _round_up(a, b):
  return (a + b - 1) // b * b


def _make_deg_kernel(Np, nch):
  mesh = plsc.VectorSubcoreMesh(core_axis_name="c", subcore_axis_name="s")
  maxcpt = -(-nch // NW)
  tp = Np // NS

  @functools.partial(
      pl.kernel,
      out_type=jax.ShapeDtypeStruct((NC * 2 * Np,), jnp.float32),
      mesh=mesh,
      compiler_params=pltpu.CompilerParams(use_tc_tiling_on_sc=False),
      scratch_types=[
          pltpu.VMEM((2 * maxcpt, CHUNK), jnp.int32),
          pltpu.VMEM((CHUNK,), jnp.float32),
          pltpu.VMEM((tp,), jnp.float32),
          pltpu.VMEM_SHARED((Np,), jnp.float32),
          pltpu.VMEM_SHARED((Np,), jnp.float32),
          pltpu.SemaphoreType.DMA,
      ],
  )
  def deg_kernel(edge_hbm, out_hbm, edge_v, ones_v, stage_v,
                 dega_sh, degb_sh, sem):
    c = lax.axis_index("c")
    s = lax.axis_index("s")
    w = c * NS + s
    cs = (nch * w) // NW
    ce = (nch * (w + 1)) // NW
    t = ce - cs
    pltpu.sync_copy(edge_hbm.at[pl.ds(2 * cs, 2 * maxcpt)], edge_v)
    for k in range(CHUNK // 16):
      ones_v[pl.ds(k * 16, 16)] = jnp.full((16,), 1.0, jnp.float32)

    def zbody(j, carry):
      stage_v[pl.ds(j * 16, 16)] = jnp.zeros((16,), jnp.float32)
      return carry

    lax.fori_loop(0, tp // 16, zbody, 0)
    off = pl.multiple_of(s * tp, 8)
    pltpu.sync_copy(stage_v, dega_sh.at[pl.ds(off, tp)])
    pltpu.sync_copy(stage_v, degb_sh.at[pl.ds(off, tp)])
    plsc.subcore_barrier()

    def body(j, carry):
      @pl.when(j < t)
      def _():
        pltpu.async_copy(ones_v, dega_sh.at[edge_v.at[2 * j]], sem, add=True)
        pltpu.async_copy(ones_v, degb_sh.at[edge_v.at[2 * j + 1]], sem,
                         add=True)

      return carry

    lax.fori_loop(0, maxcpt, body, 0)

    def drain(j, carry):
      pltpu.make_async_copy(ones_v, dega_sh.at[edge_v.at[0]], sem).wait()
      return carry

    lax.fori_loop(0, 2 * t, drain, 0)
    plsc.subcore_barrier()
    base = c * 2 * Np
    pltpu.sync_copy(dega_sh.at[pl.ds(off, tp)], stage_v)
    pltpu.sync_copy(stage_v,
                    out_hbm.at[pl.ds(pl.multiple_of(base + s * tp, 8), tp)])
    pltpu.sync_copy(degb_sh.at[pl.ds(off, tp)], stage_v)
    pltpu.sync_copy(stage_v,
                    out_hbm.at[pl.ds(pl.multiple_of(base + Np + s * tp, 8),
                                     tp)])

  return deg_kernel


def _make_prop_kernel(Np, H, nch):
  mesh = plsc.VectorSubcoreMesh(core_axis_name="c", subcore_axis_name="s")
  maxcpt = -(-nch // NW)
  rpt = Np // NS
  nq = rpt // CHUNK
  assert rpt % CHUNK == 0

  @functools.partial(
      pl.kernel,
      out_type=jax.ShapeDtypeStruct((NC * Np, H), jnp.float32),
      mesh=mesh,
      compiler_params=pltpu.CompilerParams(use_tc_tiling_on_sc=False),
      scratch_types=[
          pltpu.VMEM((2 * maxcpt, CHUNK), jnp.int32),
          pltpu.VMEM((4, CHUNK, H), jnp.float32),
          pltpu.VMEM_SHARED((Np, H), jnp.float32),
          pltpu.SemaphoreType.DMA,
          pltpu.SemaphoreType.DMA,
          pltpu.SemaphoreType.DMA,
          pltpu.SemaphoreType.DMA,
          pltpu.SemaphoreType.DMA,
          pltpu.SemaphoreType.DMA,
          pltpu.SemaphoreType.DMA,
          pltpu.SemaphoreType.DMA,
      ],
  )
  def prop_kernel(table_hbm, edge_hbm, out_hbm,
                  edge_v, rows_v, acc_sh,
                  sg0, sg1, sg2, sg3, ss0, ss1, ss2, ss3):
    c = lax.axis_index("c")
    s = lax.axis_index("s")
    w = c * NS + s
    cs = (nch * w) // NW
    ce = (nch * (w + 1)) // NW
    t = ce - cs
    sgs = (sg0, sg1, sg2, sg3)
    sss = (ss0, ss1, ss2, ss3)
    pltpu.sync_copy(edge_hbm.at[pl.ds(2 * cs, 2 * maxcpt)], edge_v)

    def zbody(j, carry):
      for k in range(H // 16):
        rows_v[0, j, pl.ds(k * 16, 16)] = jnp.zeros((16,), jnp.float32)
      return carry

    lax.fori_loop(0, CHUNK, zbody, 0)
    for q in range(nq):
      pltpu.sync_copy(rows_v.at[0],
                      acc_sh.at[pl.ds(pl.multiple_of(s * rpt + q * CHUNK, 8),
                                      CHUNK)])
    plsc.subcore_barrier()

    for b in range(2):
      @pl.when(b < t)
      def _(b=b):
        pltpu.async_copy(table_hbm.at[edge_v.at[2 * b]], rows_v.at[b], sgs[b])

    def body(jo, carry):
      for b in range(4):
        jj = 4 * jo + b
        pj = jj + 2
        pb = (b + 2) % 4

        @pl.when(pj < t)
        def _(jj=jj, pj=pj, pb=pb):
          @pl.when(pj >= 4)
          def _():
            pltpu.make_async_copy(rows_v.at[pb],
                                  acc_sh.at[edge_v.at[2 * (pj - 4) + 1]],
                                  sss[pb]).wait()

          pltpu.async_copy(table_hbm.at[edge_v.at[2 * pj]], rows_v.at[pb],
                           sgs[pb])

        @pl.when(jj < t)
        def _(jj=jj, b=b):
          pltpu.make_async_copy(table_hbm.at[edge_v.at[2 * jj]], rows_v.at[b],
                                sgs[b]).wait()
          pltpu.async_copy(rows_v.at[b], acc_sh.at[edge_v.at[2 * jj + 1]],
                           sss[b], add=True)

      return carry

    _ = lax.fori_loop(0, (maxcpt + 3) // 4, body, 0)

    for b in range(4):
      @pl.when(b < t)
      def _(b=b):
        last = ((t - 1 - b) // 4) * 4 + b
        pltpu.make_async_copy(rows_v.at[b],
                              acc_sh.at[edge_v.at[2 * last + 1]],
                              sss[b]).wait()

    plsc.subcore_barrier()

    obase = c * Np + s * rpt
    for q in range(nq):
      aq = pl.multiple_of(s * rpt + q * CHUNK, 8)
      pltpu.async_copy(acc_sh.at[pl.ds(aq, CHUNK)], rows_v.at[q % 4],
                       sgs[q % 4])
    for q in range(nq):
      aq = pl.multiple_of(s * rpt + q * CHUNK, 8)
      pltpu.make_async_copy(acc_sh.at[pl.ds(aq, CHUNK)],
                            rows_v.at[q % 4], sgs[q % 4]).wait()
      oq = pl.multiple_of(obase + q * CHUNK, 8)
      pltpu.async_copy(rows_v.at[q % 4], out_hbm.at[pl.ds(oq, CHUNK)],
                       sss[q % 4])
    for q in range(nq):
      oq = pl.multiple_of(obase + q * CHUNK, 8)
      pltpu.make_async_copy(rows_v.at[q % 4], out_hbm.at[pl.ds(oq, CHUNK)],
                            sss[q % 4]).wait()

  return prop_kernel


def _norms(deg_ref, Np, blk, i, which):
  d0 = deg_ref[pl.ds(which * Np + i * blk, blk)]
  d1 = deg_ref[pl.ds((2 + which) * Np + i * blk, blk)]
  n = lax.rsqrt(jnp.maximum(d0 + d1, 1.0))
  return jnp.reshape(n, (blk, 1))


def _prep_body(Np, blk, x_ref, w1_ref, deg_ref, g1_ref):
  i = pl.program_id(0)
  nsrc = _norms(deg_ref, Np, blk, i, 0)
  h = jnp.dot(x_ref[...], w1_ref[...], preferred_element_type=jnp.float32)
  g1_ref[...] = h * nsrc


def _mid_body(Np, blk, p_ref, deg_ref, o_ref):
  i = pl.program_id(0)
  nsrc = _norms(deg_ref, Np, blk, i, 0)
  ndst = _norms(deg_ref, Np, blk, i, 1)
  agg = p_ref[0] + p_ref[1]
  h1 = jnp.maximum(agg * ndst, 0.0)
  o_ref[...] = h1 * nsrc


def _dec_body(Np, blk, q_ref, deg_ref, w2_ref, w3_ref, mu_ref, lv_ref):
  i = pl.program_id(0)
  ndst = _norms(deg_ref, Np, blk, i, 1)
  sblk = (q_ref[0] + q_ref[1]) * ndst
  mu_ref[...] = jnp.dot(sblk, w2_ref[...], preferred_element_type=jnp.float32)
  lv_ref[...] = jnp.dot(sblk, w3_ref[...], preferred_element_type=jnp.float32)


def _adj_body(a_ref, b_ref, o_ref):
  o_ref[...] = lax.dot_general(a_ref[...], b_ref[...],
                               (((1,), (1,)), ((), ())),
                               preferred_element_type=jnp.float32)


def kernel(x, edge_index, W1, W2, W3):
  N, D = x.shape
  E = edge_index.shape[1]
  H1 = W1.shape[1]
  H2 = W2.shape[1]

  Np = _round_up(N, 2048)
  blk = 2048
  nblk = Np // blk
  nch = E // CHUNK
  assert E % CHUNK == 0

  edge_r = jnp.transpose(edge_index.reshape(2, nch, CHUNK),
                         (1, 0, 2)).reshape(2 * nch, CHUNK)

  deg_flat = _make_deg_kernel(Np, nch)(edge_r)

  x_pad = jnp.pad(x, ((0, Np - N), (0, 0)))
  g1 = pl.pallas_call(
      functools.partial(_prep_body, Np, blk),
      grid=(nblk,),
      in_specs=[
          pl.BlockSpec((blk, D), lambda i: (i, 0)),
          pl.BlockSpec((D, H1), lambda i: (0, 0)),
          pl.BlockSpec((NC * 2 * Np,), lambda i: (0,)),
      ],
      out_specs=pl.BlockSpec((blk, H1), lambda i: (i, 0)),
      out_shape=jax.ShapeDtypeStruct((Np, H1), jnp.float32),
  )(x_pad, W1, deg_flat)

  prop = _make_prop_kernel(Np, H1, nch)
  agg_parts = prop(g1, edge_r).reshape(NC, Np, H1)

  h1n = pl.pallas_call(
      functools.partial(_mid_body, Np, blk),
      grid=(nblk,),
      in_specs=[
          pl.BlockSpec((NC, blk, H1), lambda i: (0, i, 0)),
          pl.BlockSpec((NC * 2 * Np,), lambda i: (0,)),
      ],
      out_specs=pl.BlockSpec((blk, H1), lambda i: (i, 0)),
      out_shape=jax.ShapeDtypeStruct((Np, H1), jnp.float32),
  )(agg_parts, deg_flat)

  q_parts = prop(h1n, edge_r).reshape(NC, Np, H1)

  mu_pad, lv_pad = pl.pallas_call(
      functools.partial(_dec_body, Np, blk),
      grid=(nblk,),
      in_specs=[
          pl.BlockSpec((NC, blk, H1), lambda i: (0, i, 0)),
          pl.BlockSpec((NC * 2 * Np,), lambda i: (0,)),
          pl.BlockSpec((H1, H2), lambda i: (0, 0)),
          pl.BlockSpec((H1, H2), lambda i: (0, 0)),
      ],
      out_specs=[
          pl.BlockSpec((blk, H2), lambda i: (i, 0)),
          pl.BlockSpec((blk, H2), lambda i: (i, 0)),
      ],
      out_shape=[
          jax.ShapeDtypeStruct((Np, H2), jnp.float32),
          jax.ShapeDtypeStruct((Np, H2), jnp.float32),
      ],
  )(q_parts, deg_flat, W2, W3)

  mu = mu_pad[:N]
  logvar = lv_pad[:N]

  bm = 400
  adj = pl.pallas_call(
      _adj_body,
      grid=(N // bm,),
      in_specs=[
          pl.BlockSpec((bm, H2), lambda i: (i, 0)),
          pl.BlockSpec((N, H2), lambda i: (0, 0)),
      ],
      out_specs=pl.BlockSpec((bm, N), lambda i: (i, 0)),
      out_shape=jax.ShapeDtypeStruct((N, N), jnp.float32),
  )(mu, mu)

  return (adj, mu, logvar)


# speedup vs baseline: 13.3458x; 1.0534x over previous
import functools

import jax
import jax.numpy as jnp
from jax import lax
from jax.experimental import pallas as pl
from jax.experimental.pallas import tpu as pltpu
from jax.experimental.pallas import tpu_sc as plsc

NC = 2
NS = 16
NW = NC * NS
CHUNK = 128


def _round_up(a, b):
  return (a + b - 1) // b * b


def _make_deg_kernel(Np, nch):
  mesh = plsc.VectorSubcoreMesh(core_axis_name="c", subcore_axis_name="s")
  maxcpt = -(-nch // NW)
  tp = Np // NS

  @functools.partial(
      pl.kernel,
      out_type=jax.ShapeDtypeStruct((NC * 2 * Np,), jnp.float32),
      mesh=mesh,
      compiler_params=pltpu.CompilerParams(use_tc_tiling_on_sc=False),
      scratch_types=[
          pltpu.VMEM((maxcpt, 2, CHUNK), jnp.int32),
          pltpu.VMEM((CHUNK,), jnp.float32),
          pltpu.VMEM((tp,), jnp.float32),
          pltpu.VMEM_SHARED((Np,), jnp.float32),
          pltpu.VMEM_SHARED((Np,), jnp.float32),
          pltpu.SemaphoreType.DMA,
      ],
  )
  def deg_kernel(edge_hbm, out_hbm, edge_v, ones_v, stage_v,
                 dega_sh, degb_sh, sem):
    c = lax.axis_index("c")
    s = lax.axis_index("s")
    w = c * NS + s
    cs = (nch * w) // NW
    ce = (nch * (w + 1)) // NW
    t = ce - cs
    pltpu.sync_copy(edge_hbm.at[pl.ds(cs, maxcpt)], edge_v)
    for k in range(CHUNK // 16):
      ones_v[pl.ds(k * 16, 16)] = jnp.full((16,), 1.0, jnp.float32)

    def zbody(j, carry):
      stage_v[pl.ds(j * 16, 16)] = jnp.zeros((16,), jnp.float32)
      return carry

    lax.fori_loop(0, tp // 16, zbody, 0)
    off = pl.multiple_of(s * tp, 8)
    pltpu.sync_copy(stage_v, dega_sh.at[pl.ds(off, tp)])
    pltpu.sync_copy(stage_v, degb_sh.at[pl.ds(off, tp)])
    plsc.subcore_barrier()

    def body(j, carry):
      @pl.when(j < t)
      def _():
        pltpu.async_copy(ones_v, dega_sh.at[edge_v.at[j, 0]], sem, add=True)
        pltpu.async_copy(ones_v, degb_sh.at[edge_v.at[j, 1]], sem,
                         add=True)

      return carry

    lax.fori_loop(0, maxcpt, body, 0)

    def drain(j, carry):
      pltpu.make_async_copy(ones_v, dega_sh.at[edge_v.at[0, 0]], sem).wait()
      return carry

    lax.fori_loop(0, 2 * t, drain, 0)
    plsc.subcore_barrier()
    base = c * 2 * Np
    pltpu.sync_copy(dega_sh.at[pl.ds(off, tp)], stage_v)
    pltpu.sync_copy(stage_v,
                    out_hbm.at[pl.ds(pl.multiple_of(base + s * tp, 8), tp)])
    pltpu.sync_copy(degb_sh.at[pl.ds(off, tp)], stage_v)
    pltpu.sync_copy(stage_v,
                    out_hbm.at[pl.ds(pl.multiple_of(base + Np + s * tp, 8),
                                     tp)])

  return deg_kernel


def _make_prop_kernel(Np, H, nch):
  mesh = plsc.VectorSubcoreMesh(core_axis_name="c", subcore_axis_name="s")
  maxcpt = -(-nch // NW)
  rpt = Np // NS
  nq = rpt // CHUNK
  assert rpt % CHUNK == 0
  NSLOT = 6
  PD = 4

  @functools.partial(
      pl.kernel,
      out_type=jax.ShapeDtypeStruct((NC * Np, H), jnp.float32),
      mesh=mesh,
      compiler_params=pltpu.CompilerParams(use_tc_tiling_on_sc=False),
      scratch_types=[
          pltpu.VMEM((maxcpt, 2, CHUNK), jnp.int32),
          pltpu.VMEM((NSLOT, CHUNK, H), jnp.float32),
          pltpu.VMEM_SHARED((Np, H), jnp.float32),
      ] + [pltpu.SemaphoreType.DMA] * (2 * NSLOT),
  )
  def prop_kernel(table_hbm, edge_hbm, out_hbm,
                  edge_v, rows_v, acc_sh, *sems):
    c = lax.axis_index("c")
    s = lax.axis_index("s")
    w = c * NS + s
    cs = (nch * w) // NW
    ce = (nch * (w + 1)) // NW
    t = ce - cs
    sgs = sems[:NSLOT]
    sss = sems[NSLOT:]
    pltpu.sync_copy(edge_hbm.at[pl.ds(cs, maxcpt)], edge_v)

    def zbody(j, carry):
      for k in range(H // 16):
        rows_v[0, j, pl.ds(k * 16, 16)] = jnp.zeros((16,), jnp.float32)
      return carry

    lax.fori_loop(0, CHUNK, zbody, 0)
    for q in range(nq):
      pltpu.sync_copy(rows_v.at[0],
                      acc_sh.at[pl.ds(pl.multiple_of(s * rpt + q * CHUNK, 8),
                                      CHUNK)])
    plsc.subcore_barrier()

    for b in range(PD):
      @pl.when(b < t)
      def _(b=b):
        pltpu.async_copy(table_hbm.at[edge_v.at[b, 0]], rows_v.at[b], sgs[b])

    def body(jo, carry):
      for b in range(NSLOT):
        jj = NSLOT * jo + b
        pj = jj + PD
        pb = (b + PD) % NSLOT

        @pl.when(pj < t)
        def _(jj=jj, pj=pj, pb=pb):
          @pl.when(pj >= NSLOT)
          def _():
            pltpu.make_async_copy(rows_v.at[pb],
                                  acc_sh.at[edge_v.at[pj - NSLOT, 1]],
                                  sss[pb]).wait()

          pltpu.async_copy(table_hbm.at[edge_v.at[pj, 0]], rows_v.at[pb],
                           sgs[pb])

        @pl.when(jj < t)
        def _(jj=jj, b=b):
          pltpu.make_async_copy(table_hbm.at[edge_v.at[jj, 0]], rows_v.at[b],
                                sgs[b]).wait()
          pltpu.async_copy(rows_v.at[b], acc_sh.at[edge_v.at[jj, 1]],
                           sss[b], add=True)

      return carry

    _ = lax.fori_loop(0, (maxcpt + NSLOT - 1) // NSLOT, body, 0)

    for b in range(NSLOT):
      @pl.when(b < t)
      def _(b=b):
        last = ((t - 1 - b) // NSLOT) * NSLOT + b
        pltpu.make_async_copy(rows_v.at[b],
                              acc_sh.at[edge_v.at[last, 1]],
                              sss[b]).wait()

    plsc.subcore_barrier()

    obase = c * Np + s * rpt
    for q in range(nq):
      aq = pl.multiple_of(s * rpt + q * CHUNK, 8)
      pltpu.async_copy(acc_sh.at[pl.ds(aq, CHUNK)], rows_v.at[q % NSLOT],
                       sgs[q % NSLOT])
    for q in range(nq):
      aq = pl.multiple_of(s * rpt + q * CHUNK, 8)
      pltpu.make_async_copy(acc_sh.at[pl.ds(aq, CHUNK)],
                            rows_v.at[q % NSLOT], sgs[q % NSLOT]).wait()
      oq = pl.multiple_of(obase + q * CHUNK, 8)
      pltpu.async_copy(rows_v.at[q % NSLOT], out_hbm.at[pl.ds(oq, CHUNK)],
                       sss[q % NSLOT])
    for q in range(nq):
      oq = pl.multiple_of(obase + q * CHUNK, 8)
      pltpu.make_async_copy(rows_v.at[q % NSLOT],
                            out_hbm.at[pl.ds(oq, CHUNK)],
                            sss[q % NSLOT]).wait()

  return prop_kernel


def _norms(deg_ref, Np, blk, i, which):
  d0 = deg_ref[pl.ds(which * Np + i * blk, blk)]
  d1 = deg_ref[pl.ds((2 + which) * Np + i * blk, blk)]
  n = lax.rsqrt(jnp.maximum(d0 + d1, 1.0))
  return jnp.reshape(n, (blk, 1))


def _prep_body(Np, blk, x_ref, w1_ref, deg_ref, g1_ref):
  i = pl.program_id(0)
  nsrc = _norms(deg_ref, Np, blk, i, 0)
  h = jnp.dot(x_ref[...], w1_ref[...], preferred_element_type=jnp.float32)
  g1_ref[...] = h * nsrc


def _mid_body(Np, blk, p_ref, deg_ref, o_ref):
  i = pl.program_id(0)
  nsrc = _norms(deg_ref, Np, blk, i, 0)
  ndst = _norms(deg_ref, Np, blk, i, 1)
  agg = p_ref[0] + p_ref[1]
  h1 = jnp.maximum(agg * ndst, 0.0)
  o_ref[...] = h1 * nsrc


def _dec_body(Np, blk, q_ref, deg_ref, w2_ref, w3_ref, mu_ref, lv_ref):
  i = pl.program_id(0)
  ndst = _norms(deg_ref, Np, blk, i, 1)
  sblk = (q_ref[0] + q_ref[1]) * ndst
  mu_ref[...] = jnp.dot(sblk, w2_ref[...], preferred_element_type=jnp.float32)
  lv_ref[...] = jnp.dot(sblk, w3_ref[...], preferred_element_type=jnp.float32)


def _adj_body(a_ref, b_ref, o_ref):
  o_ref[...] = lax.dot_general(a_ref[...], b_ref[...],
                               (((1,), (1,)), ((), ())),
                               preferred_element_type=jnp.float32)


def kernel(x, edge_index, W1, W2, W3):
  N, D = x.shape
  E = edge_index.shape[1]
  H1 = W1.shape[1]
  H2 = W2.shape[1]

  Np = _round_up(N, 2048)
  blk = 2048
  nblk = Np // blk
  nch = E // CHUNK
  assert E % CHUNK == 0

  edge_r = jnp.transpose(edge_index.reshape(2, nch, CHUNK), (1, 0, 2))

  deg_flat = _make_deg_kernel(Np, nch)(edge_r)

  x_pad = jnp.pad(x, ((0, Np - N), (0, 0)))
  g1 = pl.pallas_call(
      functools.partial(_prep_body, Np, blk),
      grid=(nblk,),
      in_specs=[
          pl.BlockSpec((blk, D), lambda i: (i, 0)),
          pl.BlockSpec((D, H1), lambda i: (0, 0)),
          pl.BlockSpec((NC * 2 * Np,), lambda i: (0,)),
      ],
      out_specs=pl.BlockSpec((blk, H1), lambda i: (i, 0)),
      out_shape=jax.ShapeDtypeStruct((Np, H1), jnp.float32),
  )(x_pad, W1, deg_flat)

  prop = _make_prop_kernel(Np, H1, nch)
  agg_parts = prop(g1, edge_r).reshape(NC, Np, H1)

  h1n = pl.pallas_call(
      functools.partial(_mid_body, Np, blk),
      grid=(nblk,),
      in_specs=[
          pl.BlockSpec((NC, blk, H1), lambda i: (0, i, 0)),
          pl.BlockSpec((NC * 2 * Np,), lambda i: (0,)),
      ],
      out_specs=pl.BlockSpec((blk, H1), lambda i: (i, 0)),
      out_shape=jax.ShapeDtypeStruct((Np, H1), jnp.float32),
  )(agg_parts, deg_flat)

  q_parts = prop(h1n, edge_r).reshape(NC, Np, H1)

  mu_pad, lv_pad = pl.pallas_call(
      functools.partial(_dec_body, Np, blk),
      grid=(nblk,),
      in_specs=[
          pl.BlockSpec((NC, blk, H1), lambda i: (0, i, 0)),
          pl.BlockSpec((NC * 2 * Np,), lambda i: (0,)),
          pl.BlockSpec((H1, H2), lambda i: (0, 0)),
          pl.BlockSpec((H1, H2), lambda i: (0, 0)),
      ],
      out_specs=[
          pl.BlockSpec((blk, H2), lambda i: (i, 0)),
          pl.BlockSpec((blk, H2), lambda i: (i, 0)),
      ],
      out_shape=[
          jax.ShapeDtypeStruct((Np, H2), jnp.float32),
          jax.ShapeDtypeStruct((Np, H2), jnp.float32),
      ],
  )(q_parts, deg_flat, W2, W3)

  mu = mu_pad[:N]
  logvar = lv_pad[:N]

  bm = 400
  adj = pl.pallas_call(
      _adj_body,
      grid=(N // bm,),
      in_specs=[
          pl.BlockSpec((bm, H2), lambda i: (i, 0)),
          pl.BlockSpec((N, H2), lambda i: (0, 0)),
      ],
      out_specs=pl.BlockSpec((bm, N), lambda i: (i, 0)),
      out_shape=jax.ShapeDtypeStruct((N, N), jnp.float32),
  )(mu_pad, mu_pad)

  return (adj, mu, logvar)


# speedup vs baseline: 14.0919x; 1.0559x over previous
import functools

import jax
import jax.numpy as jnp
from jax import lax
from jax.experimental import pallas as pl
from jax.experimental.pallas import tpu as pltpu
from jax.experimental.pallas import tpu_sc as plsc

NC = 2
NS = 16
NW = NC * NS
CHUNK = 128


def _round_up(a, b):
  return (a + b - 1) // b * b


def _make_deg_kernel(Np, nch):
  mesh = plsc.VectorSubcoreMesh(core_axis_name="c", subcore_axis_name="s")
  maxcpt = -(-nch // NW)
  tp = Np // NS

  @functools.partial(
      pl.kernel,
      out_type=jax.ShapeDtypeStruct((NC * 2 * Np,), jnp.float32),
      mesh=mesh,
      compiler_params=pltpu.CompilerParams(use_tc_tiling_on_sc=False),
      scratch_types=[
          pltpu.VMEM((maxcpt, 2, CHUNK), jnp.int32),
          pltpu.VMEM((CHUNK,), jnp.float32),
          pltpu.VMEM((tp,), jnp.float32),
          pltpu.VMEM_SHARED((Np,), jnp.float32),
          pltpu.VMEM_SHARED((Np,), jnp.float32),
          pltpu.SemaphoreType.DMA,
      ],
  )
  def deg_kernel(edge_hbm, out_hbm, edge_v, ones_v, stage_v,
                 dega_sh, degb_sh, sem):
    c = lax.axis_index("c")
    s = lax.axis_index("s")
    w = c * NS + s
    cs = (nch * w) // NW
    ce = (nch * (w + 1)) // NW
    t = ce - cs
    pltpu.sync_copy(edge_hbm.at[pl.ds(cs, maxcpt)], edge_v)
    for k in range(CHUNK // 16):
      ones_v[pl.ds(k * 16, 16)] = jnp.full((16,), 1.0, jnp.float32)

    def zbody(j, carry):
      stage_v[pl.ds(j * 16, 16)] = jnp.zeros((16,), jnp.float32)
      return carry

    lax.fori_loop(0, tp // 16, zbody, 0)
    off = pl.multiple_of(s * tp, 8)
    pltpu.sync_copy(stage_v, dega_sh.at[pl.ds(off, tp)])
    pltpu.sync_copy(stage_v, degb_sh.at[pl.ds(off, tp)])
    plsc.subcore_barrier()

    def body(j, carry):
      @pl.when(j < t)
      def _():
        pltpu.async_copy(ones_v, dega_sh.at[edge_v.at[j, 0]], sem, add=True)
        pltpu.async_copy(ones_v, degb_sh.at[edge_v.at[j, 1]], sem,
                         add=True)

      return carry

    lax.fori_loop(0, maxcpt, body, 0)

    def drain(j, carry):
      pltpu.make_async_copy(ones_v, dega_sh.at[edge_v.at[0, 0]], sem).wait()
      return carry

    lax.fori_loop(0, 2 * t, drain, 0)
    plsc.subcore_barrier()
    base = c * 2 * Np
    pltpu.sync_copy(dega_sh.at[pl.ds(off, tp)], stage_v)
    pltpu.sync_copy(stage_v,
                    out_hbm.at[pl.ds(pl.multiple_of(base + s * tp, 8), tp)])
    pltpu.sync_copy(degb_sh.at[pl.ds(off, tp)], stage_v)
    pltpu.sync_copy(stage_v,
                    out_hbm.at[pl.ds(pl.multiple_of(base + Np + s * tp, 8),
                                     tp)])

  return deg_kernel


def _make_prop_kernel(Np, H, nch):
  mesh = plsc.VectorSubcoreMesh(core_axis_name="c", subcore_axis_name="s")
  maxcpt = -(-nch // NW)
  rpt = Np // NS
  nq = rpt // CHUNK
  assert rpt % CHUNK == 0
  NSLOT = 6
  PD = 4

  @functools.partial(
      pl.kernel,
      out_type=jax.ShapeDtypeStruct((NC * Np, H), jnp.float32),
      mesh=mesh,
      compiler_params=pltpu.CompilerParams(use_tc_tiling_on_sc=False),
      scratch_types=[
          pltpu.VMEM((maxcpt, 2, CHUNK), jnp.int32),
          pltpu.VMEM((NSLOT, CHUNK, H), jnp.float32),
          pltpu.VMEM_SHARED((Np, H), jnp.float32),
      ] + [pltpu.SemaphoreType.DMA] * (2 * NSLOT),
  )
  def prop_kernel(table_hbm, edge_hbm, out_hbm,
                  edge_v, rows_v, acc_sh, *sems):
    c = lax.axis_index("c")
    s = lax.axis_index("s")
    w = c * NS + s
    cs = (nch * w) // NW
    ce = (nch * (w + 1)) // NW
    t = ce - cs
    sgs = sems[:NSLOT]
    sss = sems[NSLOT:]
    pltpu.sync_copy(edge_hbm.at[pl.ds(cs, maxcpt)], edge_v)

    def dbl(j, carry):
      for k in range(CHUNK // 16):
        v = edge_v[j, 0, pl.ds(k * 16, 16)]
        edge_v[j, 0, pl.ds(k * 16, 16)] = v + v
      return carry

    lax.fori_loop(0, maxcpt, dbl, 0)

    def zbody(j, carry):
      for k in range(H // 16):
        rows_v[0, j, pl.ds(k * 16, 16)] = jnp.zeros((16,), jnp.float32)
      return carry

    lax.fori_loop(0, CHUNK, zbody, 0)
    for q in range(nq):
      pltpu.sync_copy(rows_v.at[0],
                      acc_sh.at[pl.ds(pl.multiple_of(s * rpt + q * CHUNK, 8),
                                      CHUNK)])
    plsc.subcore_barrier()

    for b in range(PD):
      @pl.when(b < t)
      def _(b=b):
        pltpu.async_copy(table_hbm.at[edge_v.at[b, 0]], rows_v.at[b], sgs[b])

    def body(jo, carry):
      for b in range(NSLOT):
        jj = NSLOT * jo + b
        pj = jj + PD
        pb = (b + PD) % NSLOT

        @pl.when(pj < t)
        def _(jj=jj, pj=pj, pb=pb):
          @pl.when(pj >= NSLOT)
          def _():
            pltpu.make_async_copy(rows_v.at[pb],
                                  acc_sh.at[edge_v.at[pj - NSLOT, 1]],
                                  sss[pb]).wait()

          pltpu.async_copy(table_hbm.at[edge_v.at[pj, 0]], rows_v.at[pb],
                           sgs[pb])

        @pl.when(jj < t)
        def _(jj=jj, b=b):
          pltpu.make_async_copy(table_hbm.at[edge_v.at[jj, 0]], rows_v.at[b],
                                sgs[b]).wait()
          pltpu.async_copy(rows_v.at[b], acc_sh.at[edge_v.at[jj, 1]],
                           sss[b], add=True)

      return carry

    _ = lax.fori_loop(0, (maxcpt + NSLOT - 1) // NSLOT, body, 0)

    for b in range(NSLOT):
      @pl.when(b < t)
      def _(b=b):
        last = ((t - 1 - b) // NSLOT) * NSLOT + b
        pltpu.make_async_copy(rows_v.at[b],
                              acc_sh.at[edge_v.at[last, 1]],
                              sss[b]).wait()

    plsc.subcore_barrier()

    obase = c * Np + s * rpt
    for q in range(nq):
      aq = pl.multiple_of(s * rpt + q * CHUNK, 8)
      pltpu.async_copy(acc_sh.at[pl.ds(aq, CHUNK)], rows_v.at[q % NSLOT],
                       sgs[q % NSLOT])
    for q in range(nq):
      aq = pl.multiple_of(s * rpt + q * CHUNK, 8)
      pltpu.make_async_copy(acc_sh.at[pl.ds(aq, CHUNK)],
                            rows_v.at[q % NSLOT], sgs[q % NSLOT]).wait()
      oq = pl.multiple_of(obase + q * CHUNK, 8)
      pltpu.async_copy(rows_v.at[q % NSLOT], out_hbm.at[pl.ds(oq, CHUNK)],
                       sss[q % NSLOT])
    for q in range(nq):
      oq = pl.multiple_of(obase + q * CHUNK, 8)
      pltpu.make_async_copy(rows_v.at[q % NSLOT],
                            out_hbm.at[pl.ds(oq, CHUNK)],
                            sss[q % NSLOT]).wait()

  return prop_kernel


def _norms(deg_ref, Np, blk, i, which):
  d0 = deg_ref[pl.ds(which * Np + i * blk, blk)]
  d1 = deg_ref[pl.ds((2 + which) * Np + i * blk, blk)]
  n = lax.rsqrt(jnp.maximum(d0 + d1, 1.0))
  return jnp.reshape(n, (blk, 1))


def _prep_body(Np, blk, x_ref, w1_ref, deg_ref, g1_ref):
  i = pl.program_id(0)
  nsrc = _norms(deg_ref, Np, blk, i, 0)
  h = jnp.dot(x_ref[...], w1_ref[...], preferred_element_type=jnp.float32)
  g = h * nsrc
  g1_ref[...] = jnp.concatenate([g, jnp.zeros_like(g)], axis=1)


def _mid_body(Np, blk, p_ref, deg_ref, o_ref):
  i = pl.program_id(0)
  nsrc = _norms(deg_ref, Np, blk, i, 0)
  ndst = _norms(deg_ref, Np, blk, i, 1)
  agg = p_ref[0] + p_ref[1]
  h1 = jnp.maximum(agg * ndst, 0.0)
  h = h1 * nsrc
  o_ref[...] = jnp.concatenate([h, jnp.zeros_like(h)], axis=1)


def _dec_body(Np, blk, q_ref, deg_ref, w2_ref, w3_ref, mu_ref, lv_ref):
  i = pl.program_id(0)
  ndst = _norms(deg_ref, Np, blk, i, 1)
  sblk = (q_ref[0] + q_ref[1]) * ndst
  mu_ref[...] = jnp.dot(sblk, w2_ref[...], preferred_element_type=jnp.float32)
  lv_ref[...] = jnp.dot(sblk, w3_ref[...], preferred_element_type=jnp.float32)


def _adj_body(a_ref, b_ref, o_ref):
  o_ref[...] = lax.dot_general(a_ref[...], b_ref[...],
                               (((1,), (1,)), ((), ())),
                               preferred_element_type=jnp.float32)


def kernel(x, edge_index, W1, W2, W3):
  N, D = x.shape
  E = edge_index.shape[1]
  H1 = W1.shape[1]
  H2 = W2.shape[1]

  Np = _round_up(N, 2048)
  blk = 2048
  nblk = Np // blk
  nch = E // CHUNK
  assert E % CHUNK == 0

  edge_r = jnp.transpose(edge_index.reshape(2, nch, CHUNK), (1, 0, 2))

  deg_flat = _make_deg_kernel(Np, nch)(edge_r)

  x_pad = jnp.pad(x, ((0, Np - N), (0, 0)))
  g1 = pl.pallas_call(
      functools.partial(_prep_body, Np, blk),
      grid=(nblk,),
      in_specs=[
          pl.BlockSpec((blk, D), lambda i: (i, 0)),
          pl.BlockSpec((D, H1), lambda i: (0, 0)),
          pl.BlockSpec((NC * 2 * Np,), lambda i: (0,)),
      ],
      out_specs=pl.BlockSpec((blk, 2 * H1), lambda i: (i, 0)),
      out_shape=jax.ShapeDtypeStruct((Np, 2 * H1), jnp.float32),
  )(x_pad, W1, deg_flat)
  g1t = g1.reshape(2 * Np, H1)

  prop = _make_prop_kernel(Np, H1, nch)
  agg_parts = prop(g1t, edge_r).reshape(NC, Np, H1)

  h1n = pl.pallas_call(
      functools.partial(_mid_body, Np, blk),
      grid=(nblk,),
      in_specs=[
          pl.BlockSpec((NC, blk, H1), lambda i: (0, i, 0)),
          pl.BlockSpec((NC * 2 * Np,), lambda i: (0,)),
      ],
      out_specs=pl.BlockSpec((blk, 2 * H1), lambda i: (i, 0)),
      out_shape=jax.ShapeDtypeStruct((Np, 2 * H1), jnp.float32),
  )(agg_parts, deg_flat)
  h1nt = h1n.reshape(2 * Np, H1)

  q_parts = prop(h1nt, edge_r).reshape(NC, Np, H1)

  mu, logvar = pl.pallas_call(
      functools.partial(_dec_body, Np, blk),
      grid=(nblk,),
      in_specs=[
          pl.BlockSpec((NC, blk, H1), lambda i: (0, i, 0)),
          pl.BlockSpec((NC * 2 * Np,), lambda i: (0,)),
          pl.BlockSpec((H1, H2), lambda i: (0, 0)),
          pl.BlockSpec((H1, H2), lambda i: (0, 0)),
      ],
      out_specs=[
          pl.BlockSpec((blk, H2), lambda i: (i, 0)),
          pl.BlockSpec((blk, H2), lambda i: (i, 0)),
      ],
      out_shape=[
          jax.ShapeDtypeStruct((N, H2), jnp.float32),
          jax.ShapeDtypeStruct((N, H2), jnp.float32),
      ],
  )(q_parts, deg_flat, W2, W3)

  bm = 400
  adj = pl.pallas_call(
      _adj_body,
      grid=(N // bm,),
      in_specs=[
          pl.BlockSpec((bm, H2), lambda i: (i, 0)),
          pl.BlockSpec((N, H2), lambda i: (0, 0)),
      ],
      out_specs=pl.BlockSpec((bm, N), lambda i: (i, 0)),
      out_shape=jax.ShapeDtypeStruct((N, N), jnp.float32),
  )(mu, mu)

  return (adj, mu, logvar)
